# baseline (speedup 1.0000x reference)
import functools
import math

import jax
import jax.numpy as jnp
from jax import lax
from jax.experimental import pallas as pl
from jax.experimental.pallas import tpu as pltpu
from jax.experimental.pallas import tpu_sc as plsc

_NC = 2
_NS = 16
_NW = _NC * _NS
_LANES = 16
_G = 100
_NBUF = 4


def _sc_embed(idx2d, token_table, pos_table, *, batch, seq, dim, scale):
  ngroups = idx2d.shape[0]
  gpw = ngroups // _NW
  nq = gpw // _NBUF
  mesh = plsc.VectorSubcoreMesh(
      core_axis_name="c", subcore_axis_name="s",
      num_cores=_NC, num_subcores=_NS)

  @functools.partial(
      pl.kernel,
      mesh=mesh,
      out_type=jax.ShapeDtypeStruct((ngroups, _G, dim), jnp.float32),
      scratch_types=(
          [pltpu.VMEM((gpw, _G), jnp.int32),
           pltpu.VMEM((_NBUF, _G, dim), jnp.float32),
           pltpu.VMEM((seq, dim), jnp.float32)]
          + [pltpu.SemaphoreType.DMA] * (2 * _NBUF)
      ),
  )
  def k(idx_hbm, table_hbm, pos_hbm, out_hbm, idx_v, rows_v, pos_v, *sems):
    sem_g = sems[:_NBUF]
    sem_w = sems[_NBUF:]
    wid = lax.axis_index("s") * _NC + lax.axis_index("c")
    gbase = wid * gpw
    pltpu.sync_copy(pos_hbm, pos_v)
    pltpu.sync_copy(idx_hbm.at[pl.ds(gbase, gpw)], idx_v)

    def gather(g, b):
      pltpu.async_copy(table_hbm.at[idx_v.at[g]], rows_v.at[b], sem_g[b])

    def wait_gather(g, b):
      pltpu.make_async_copy(
          table_hbm.at[idx_v.at[g]], rows_v.at[b], sem_g[b]).wait()

    def wait_write(b):
      pltpu.make_async_copy(
          rows_v.at[b], out_hbm.at[gbase], sem_w[b]).wait()

    gather(0, 0)
    gather(1, 1)

    def quad(q, carry):
      for b in range(_NBUF):
        g = q * _NBUF + b
        wait_gather(g, b)

        bn = (b + 2) % _NBUF
        if b < 2:
          @pl.when(q > 0)
          def _():
            wait_write(bn)
          gather(g + 2, bn)
        else:
          wait_write(bn)

          @pl.when(q < nq - 1)
          def _():
            gather(g + 2, bn)

        phase = (b % 2) * _G

        def fma(l, c, _b=b, _ph=phase):
          for d in range(dim // _LANES):
            sl = pl.ds(d * _LANES, _LANES)
            rows_v[_b, l, sl] = rows_v[_b, l, sl] * scale + pos_v[_ph + l, sl]
          return c
        lax.fori_loop(0, _G, fma, 0)

        pltpu.async_copy(rows_v.at[b], out_hbm.at[gbase + g], sem_w[b])
      return carry

    lax.fori_loop(0, nq, quad, 0)

    for b in (2, 3):
      wait_write(b)

  out = k(idx2d, token_table, pos_table)
  return out.reshape(batch, seq, dim)


def kernel(inputs, token_table, pos_table):
  batch, seq = inputs.shape
  vocab, dim = token_table.shape
  scale = float(math.sqrt(dim))
  idx2d = inputs.reshape(batch * seq // _G, _G)
  return _sc_embed(idx2d, token_table, pos_table,
                   batch=batch, seq=seq, dim=dim, scale=scale)


# speedup vs baseline: 1.9473x; 1.9473x over previous
import functools
import math

import jax
import jax.numpy as jnp
from jax import lax
from jax.experimental import pallas as pl
from jax.experimental.pallas import tpu as pltpu
from jax.experimental.pallas import tpu_sc as plsc

_NC = 2
_NS = 16
_NW = _NC * _NS
_LANES = 16
_G = 104
_NBUF = 4


def _sc_embed(idx2, token_table, pos_table, *, batch, seq, dim, scale):
  rpw = batch // _NW
  nq = 2 * rpw // _NBUF
  ov = 2 * _G - seq
  wgeom = ((0, 0, _G), (ov, _G, seq - _G))
  pgeom = (0, seq - _G)
  mesh = plsc.VectorSubcoreMesh(
      core_axis_name="c", subcore_axis_name="s",
      num_cores=_NC, num_subcores=_NS)

  @functools.partial(
      pl.kernel,
      mesh=mesh,
      out_type=jax.ShapeDtypeStruct((batch, seq, dim), jnp.float32),
      scratch_types=(
          [pltpu.VMEM((2 * rpw, _G), jnp.int32),
           pltpu.VMEM((_NBUF, _G, dim), jnp.float32),
           pltpu.VMEM((seq, dim), jnp.float32)]
          + [pltpu.SemaphoreType.DMA] * (2 * _NBUF)
      ),
  )
  def k(idx_hbm, table_hbm, pos_hbm, out_hbm, idx_v, rows_v, pos_v, *sems):
    sem_g = sems[:_NBUF]
    sem_w = sems[_NBUF:]
    wid = lax.axis_index("s") * _NC + lax.axis_index("c")
    rbase = wid * rpw
    pltpu.sync_copy(pos_hbm, pos_v)
    pltpu.sync_copy(idx_hbm.at[pl.ds(2 * rbase, 2 * rpw)], idx_v)

    def gather(u, b):
      pltpu.async_copy(table_hbm.at[idx_v.at[u]], rows_v.at[b], sem_g[b])

    def wait_gather(b):
      pltpu.make_async_copy(
          table_hbm.at[idx_v.at[0]], rows_v.at[b], sem_g[b]).wait()

    def wait_write(half, b):
      vo, oo, n = wgeom[half]
      pltpu.make_async_copy(
          rows_v.at[b, pl.ds(vo, n)],
          out_hbm.at[0, pl.ds(oo, n)], sem_w[b]).wait()

    gather(0, 0)
    gather(1, 1)

    def quad(q, carry):
      for b in range(_NBUF):
        u = q * _NBUF + b
        half = b % 2
        wait_gather(b)

        bn = (b + 2) % _NBUF
        if b < 2:
          @pl.when(q > 0)
          def _():
            wait_write(half, bn)
          gather(u + 2, bn)
        else:
          wait_write(half, bn)

          @pl.when(q < nq - 1)
          def _():
            gather(u + 2, bn)

        def fma(l, c, _b=b, _ph=pgeom[half]):
          for d in range(dim // _LANES):
            sl = pl.ds(d * _LANES, _LANES)
            rows_v[_b, l, sl] = rows_v[_b, l, sl] * scale + pos_v[_ph + l, sl]
          return c
        lax.fori_loop(0, _G, fma, 0)

        vo, oo, n = wgeom[half]
        pltpu.async_copy(
            rows_v.at[b, pl.ds(vo, n)],
            out_hbm.at[rbase + q * 2 + b // 2, pl.ds(oo, n)],
            sem_w[b])
      return carry

    lax.fori_loop(0, nq, quad, 0)

    for b in (2, 3):
      wait_write(b % 2, b)

  return k(idx2, token_table, pos_table)


def kernel(inputs, token_table, pos_table):
  batch, seq = inputs.shape
  vocab, dim = token_table.shape
  scale = float(math.sqrt(dim))
  idx2 = jnp.stack([inputs[:, :_G], inputs[:, seq - _G:]], axis=1)
  idx2 = idx2.reshape(2 * batch, _G)
  return _sc_embed(idx2, token_table, pos_table,
                   batch=batch, seq=seq, dim=dim, scale=scale)
